# Initial kernel scaffold; baseline (speedup 1.0000x reference)
import jax
import jax.numpy as jnp
from jax.experimental import pallas as pl


def kernel(x, edge_index, W1, b1, W2, b2, Wl, bl):
    raise NotImplementedError("write your pallas kernel here")


# speedup vs baseline: 10.6638x; 10.6638x over previous
import functools

import jax
import jax.numpy as jnp
from jax import lax
from jax.experimental import pallas as pl
from jax.experimental.pallas import tpu as pltpu
from jax.experimental.pallas import tpu_sc as plsc

N_NODES = 10000
N_EDGES = 320000
D = 128

NCORES = 2
NSUB = 16
NW = NCORES * NSUB

CHUNK = 128
NCHUNK = 79
EPW = NCHUNK * CHUNK
EPAD = NW * EPW

NPAD = 10240
DUMMY = N_NODES + 64
RPS = NPAD // NSUB
RCH = RPS // CHUNK


def _deg_body(dst_hbm, ones_hbm, zeros_hbm, out_hbm, idx_v, row_v, deg_sh, sem):
    c = lax.axis_index("c")
    s = lax.axis_index("s")
    wid = c * NSUB + s

    pltpu.sync_copy(zeros_hbm, row_v)
    for j in range(RCH):
        pltpu.sync_copy(row_v, deg_sh.at[pl.ds(s * RPS + j * CHUNK, CHUNK)])
    plsc.subcore_barrier()

    pltpu.sync_copy(ones_hbm, row_v)

    def body(i, carry):
        pltpu.sync_copy(dst_hbm.at[wid, i], idx_v)
        pltpu.sync_copy(row_v, deg_sh.at[idx_v], add=True)
        return carry

    lax.fori_loop(0, NCHUNK, body, 0)
    plsc.subcore_barrier()

    for j in range(RCH):
        r = s * RPS + j * CHUNK
        pltpu.sync_copy(deg_sh.at[pl.ds(r, CHUNK)], row_v)
        pltpu.sync_copy(row_v, out_hbm.at[c, pl.ds(r, CHUNK)])


def _agg_body(g_hbm, src_hbm, dst_hbm, zeros_hbm, out_hbm,
              src_v, dst_v, rows_v, acc_sh, sem):
    c = lax.axis_index("c")
    s = lax.axis_index("s")
    wid = c * NSUB + s

    pltpu.sync_copy(zeros_hbm, rows_v)
    for j in range(RCH):
        pltpu.sync_copy(rows_v, acc_sh.at[pl.ds(s * RPS + j * CHUNK, CHUNK)])
    plsc.subcore_barrier()

    def body(i, carry):
        pltpu.sync_copy(src_hbm.at[wid, i], src_v)
        pltpu.sync_copy(dst_hbm.at[wid, i], dst_v)
        pltpu.async_copy(g_hbm.at[src_v], rows_v, sem).wait()
        pltpu.sync_copy(rows_v, acc_sh.at[dst_v], add=True)
        return carry

    lax.fori_loop(0, NCHUNK, body, 0)
    plsc.subcore_barrier()

    for j in range(RCH):
        r = s * RPS + j * CHUNK
        pltpu.sync_copy(acc_sh.at[pl.ds(r, CHUNK)], rows_v)
        pltpu.sync_copy(rows_v, out_hbm.at[c, pl.ds(r, CHUNK)])


@functools.lru_cache(maxsize=None)
def _sc_kernels():
    mesh = plsc.VectorSubcoreMesh(
        core_axis_name="c", subcore_axis_name="s",
        num_cores=NCORES, num_subcores=NSUB)
    deg_sc = pl.kernel(
        _deg_body,
        mesh=mesh,
        out_type=jax.ShapeDtypeStruct((NCORES, NPAD, D), jnp.float32),
        scratch_types=[
            pltpu.VMEM((CHUNK,), jnp.int32),
            pltpu.VMEM((CHUNK, D), jnp.float32),
            pltpu.VMEM_SHARED((NPAD, D), jnp.float32),
            pltpu.SemaphoreType.DMA,
        ],
    )
    agg_sc = pl.kernel(
        _agg_body,
        mesh=mesh,
        out_type=jax.ShapeDtypeStruct((NCORES, NPAD, D), jnp.float32),
        scratch_types=[
            pltpu.VMEM((CHUNK,), jnp.int32),
            pltpu.VMEM((CHUNK,), jnp.int32),
            pltpu.VMEM((CHUNK, D), jnp.float32),
            pltpu.VMEM_SHARED((NPAD, D), jnp.float32),
            pltpu.SemaphoreType.DMA,
        ],
    )
    return deg_sc, agg_sc


def _dinv(deg2_ref):
    deg = 1.0 + deg2_ref[0, :N_NODES, 0:1] + deg2_ref[1, :N_NODES, 0:1]
    return lax.rsqrt(deg)


def _tc1_body(x_ref, w1_ref, deg2_ref, g1_ref):
    h = jnp.dot(x_ref[...], w1_ref[...], preferred_element_type=jnp.float32)
    g1_ref[...] = h * _dinv(deg2_ref)


def _tc2_body(acc_ref, g1_ref, deg2_ref, w2_ref, b1_ref, g2_ref):
    dinv = _dinv(deg2_ref)
    agg = acc_ref[0, :N_NODES, :] + acc_ref[1, :N_NODES, :] + g1_ref[...]
    z = jnp.maximum(agg * dinv + b1_ref[...], 0.0)
    g2_ref[...] = jnp.dot(z, w2_ref[...], preferred_element_type=jnp.float32) * dinv


def _tc3_body(acc_ref, g2_ref, deg2_ref, b2_ref, wl_ref, bl_ref, out_ref):
    dinv = _dinv(deg2_ref)
    agg = acc_ref[0, :N_NODES, :] + acc_ref[1, :N_NODES, :] + g2_ref[...]
    z = jnp.maximum(agg * dinv + b2_ref[...], 0.0)
    out_ref[...] = jnp.dot(z, wl_ref[...], preferred_element_type=jnp.float32) + bl_ref[...]


_tc1 = pl.pallas_call(
    _tc1_body,
    out_shape=jax.ShapeDtypeStruct((N_NODES, D), jnp.float32),
)

_tc2 = pl.pallas_call(
    _tc2_body,
    out_shape=jax.ShapeDtypeStruct((N_NODES, D), jnp.float32),
)

_tc3 = pl.pallas_call(
    _tc3_body,
    out_shape=jax.ShapeDtypeStruct((N_NODES, D), jnp.float32),
)


@jax.jit
def kernel(x, edge_index, W1, b1, W2, b2, Wl, bl):
    src = edge_index[0].astype(jnp.int32)
    dst = edge_index[1].astype(jnp.int32)
    src_p = jnp.concatenate(
        [src, jnp.zeros((EPAD - N_EDGES,), jnp.int32)]).reshape(NW, NCHUNK, CHUNK)
    dst_p = jnp.concatenate(
        [dst, jnp.full((EPAD - N_EDGES,), DUMMY, jnp.int32)]).reshape(NW, NCHUNK, CHUNK)

    onesD = jnp.ones((CHUNK, D), jnp.float32)
    zerosD = jnp.zeros((CHUNK, D), jnp.float32)

    _deg_sc, _agg_sc = _sc_kernels()
    deg2 = _deg_sc(dst_p, onesD, zerosD)[:, :, :16]

    g1 = _tc1(x, W1, deg2)
    acc1 = _agg_sc(g1, src_p, dst_p, zerosD)

    b1r = jnp.broadcast_to(b1.reshape(1, D), (1, D))
    g2 = _tc2(acc1, g1, deg2, W2, b1r)
    acc2 = _agg_sc(g2, src_p, dst_p, zerosD)

    wl_pad = jnp.zeros((D, D), jnp.float32).at[:, : Wl.shape[1]].set(Wl)
    bl_pad = jnp.zeros((1, D), jnp.float32).at[0, : bl.shape[0]].set(bl)
    b2r = jnp.broadcast_to(b2.reshape(1, D), (1, D))
    out_pad = _tc3(acc2, g2, deg2, b2r, wl_pad, bl_pad)
    return out_pad[:, : Wl.shape[1]]
